# baseline (speedup 1.0000x reference)
import functools

import jax
import jax.numpy as jnp
from jax import lax
from jax.experimental import pallas as pl
from jax.experimental.pallas import tpu as pltpu
from jax.experimental.pallas import tpu_sc as plsc

_K = 8192
_D = 256
_ROW_TILE = 256
_TMPL_ROWS = 8
_GAMMA = 1.0


def _vq_kernel(x_ref, cb_ref, idx_ref, tmpl_ref, cbn_ref, hc_acc, div_acc):
    i = pl.program_id(0)
    nsteps = pl.num_programs(0)
    n_rows = nsteps * _ROW_TILE

    x = x_ref[...]
    cb = cb_ref[...]
    sim = jax.lax.dot_general(
        x, cb, (((1,), (1,)), ((), ())),
        preferred_element_type=jnp.float32,
        precision=jax.lax.Precision.DEFAULT)

    m = jnp.max(sim, axis=1, keepdims=True)
    e = jnp.exp(sim - m)
    z = jnp.sum(e, axis=1, keepdims=True)

    ws = jnp.sum(e * sim, axis=1, keepdims=True)
    hc_tile = jnp.sum(ws / z - m - jnp.log(z)) * (1.0 / jnp.log(2.0))
    div_tile = jax.lax.dot_general(
        1.0 / z, e, (((0,), (0,)), ((), ())),
        preferred_element_type=jnp.float32,
        precision=jax.lax.Precision.DEFAULT)

    idx = jnp.argmax(sim, axis=1).astype(jnp.int32)
    idx_ref[0, 0, :] = idx

    @pl.when(i == 0)
    def _():
        cbn = cb - jnp.mean(cb, axis=1, keepdims=True)
        cbn_ref[...] = cbn / jnp.linalg.norm(cbn, axis=1, keepdims=True)
        hc_acc[0, 0] = hc_tile
        div_acc[...] = div_tile

    @pl.when(i > 0)
    def _():
        hc_acc[0, 0] += hc_tile
        div_acc[...] += div_tile

    @pl.when(i == nsteps - 1)
    def _():
        h_clust = -(hc_acc[0, 0] / n_rows)
        div = div_acc[...] / n_rows
        h_div = -jnp.sum(div * jnp.log2(div + 1e-8))
        loss = h_clust - _GAMMA * h_div
        tmpl_ref[...] = jnp.full(tmpl_ref.shape, loss, dtype=jnp.float32)


def _make_sc_fill(n_rows):
    info = plsc.get_sparse_core_info()
    nw = info.num_cores * info.num_subcores
    b_per_w = n_rows // nw
    n_chunks = b_per_w // _TMPL_ROWS
    mesh = plsc.VectorSubcoreMesh(core_axis_name="c", subcore_axis_name="s")

    @functools.partial(
        pl.kernel, mesh=mesh,
        out_type=jax.ShapeDtypeStruct((n_rows, _K), jnp.float32),
        scratch_types=[
            pltpu.VMEM((_TMPL_ROWS, _K), jnp.float32),
        ],
    )
    def fill(tmpl_hbm, out_hbm, tmpl_v):
        wid = lax.axis_index("s") * info.num_cores + lax.axis_index("c")
        base = wid * b_per_w
        pltpu.sync_copy(tmpl_hbm, tmpl_v)
        for j in range(n_chunks):
            pltpu.sync_copy(
                tmpl_v, out_hbm.at[pl.ds(base + _TMPL_ROWS * j, _TMPL_ROWS)])

    return fill


def _make_sc_gather(n_rows):
    info = plsc.get_sparse_core_info()
    nw = info.num_cores * info.num_subcores
    b_per_w = n_rows // nw
    mesh = plsc.VectorSubcoreMesh(core_axis_name="c", subcore_axis_name="s")

    @functools.partial(
        pl.kernel, mesh=mesh,
        out_type=jax.ShapeDtypeStruct((n_rows, _D), jnp.float32),
        scratch_types=[
            pltpu.VMEM((b_per_w,), jnp.int32),
            pltpu.VMEM((b_per_w, _D), jnp.float32),
            pltpu.SemaphoreType.DMA,
        ],
    )
    def gather(table_hbm, idx_hbm, out_hbm, idx_v, rows_v, sem):
        wid = lax.axis_index("s") * info.num_cores + lax.axis_index("c")
        base = wid * b_per_w
        pltpu.sync_copy(idx_hbm.at[pl.ds(base, b_per_w)], idx_v)
        pltpu.async_copy(table_hbm.at[idx_v], rows_v, sem).wait()
        pltpu.sync_copy(rows_v, out_hbm.at[pl.ds(base, b_per_w)])

    return gather


def kernel(inputs, codebook):
    b, t, d = inputs.shape
    n = b * t
    nsteps = n // _ROW_TILE
    x = inputs.reshape(n, d)

    idx3, tmpl, cbn = pl.pallas_call(
        _vq_kernel,
        grid=(nsteps,),
        in_specs=[
            pl.BlockSpec((_ROW_TILE, _D), lambda i: (i, 0)),
            pl.BlockSpec((_K, _D), lambda i: (0, 0)),
        ],
        out_specs=[
            pl.BlockSpec((1, 1, _ROW_TILE), lambda i: (i, 0, 0)),
            pl.BlockSpec((_TMPL_ROWS, _K), lambda i: (0, 0)),
            pl.BlockSpec((_K, _D), lambda i: (0, 0)),
        ],
        out_shape=[
            jax.ShapeDtypeStruct((nsteps, 1, _ROW_TILE), jnp.int32),
            jax.ShapeDtypeStruct((_TMPL_ROWS, _K), jnp.float32),
            jax.ShapeDtypeStruct((_K, _D), jnp.float32),
        ],
        scratch_shapes=[
            pltpu.SMEM((1, 1), jnp.float32),
            pltpu.VMEM((1, _K), jnp.float32),
        ],
        compiler_params=pltpu.CompilerParams(
            dimension_semantics=("arbitrary",)),
    )(x, codebook)

    idx_flat = idx3.reshape(n)
    quant = _make_sc_gather(n)(cbn, idx_flat)
    qloss = _make_sc_fill(n)(tmpl)

    quantized = quant.reshape(1, b, t, d)
    nn_idx = idx3.reshape(b, t)
    return (quantized, qloss.reshape(b, t, _K), nn_idx, codebook)


# speedup vs baseline: 1.1215x; 1.1215x over previous
import functools

import jax
import jax.numpy as jnp
from jax import lax
from jax.experimental import pallas as pl
from jax.experimental.pallas import tpu as pltpu
from jax.experimental.pallas import tpu_sc as plsc

_K = 8192
_D = 256
_ROW_TILE = 384
_TMPL_ROWS = 8
_GAMMA = 1.0


def _vq_kernel(x_ref, cb_ref, idx_ref, loss_ref, cbn_ref, hc_acc, div_acc):
    i = pl.program_id(0)
    nsteps = pl.num_programs(0)
    n_rows = nsteps * _ROW_TILE

    x = x_ref[...]
    cb = cb_ref[...]
    sim = jax.lax.dot_general(
        x, cb, (((1,), (1,)), ((), ())),
        preferred_element_type=jnp.float32,
        precision=jax.lax.Precision.DEFAULT)

    m = jnp.max(sim, axis=1, keepdims=True)
    e = jnp.exp(sim - m)
    z = jnp.sum(e, axis=1, keepdims=True)

    ws = jnp.sum(e * sim, axis=1, keepdims=True)
    hc_tile = jnp.sum(ws / z - m - jnp.log(z)) * (1.0 / jnp.log(2.0))
    div_tile = jax.lax.dot_general(
        1.0 / z, e, (((0,), (0,)), ((), ())),
        preferred_element_type=jnp.float32,
        precision=jax.lax.Precision.DEFAULT)

    idx = jnp.argmax(sim, axis=1).astype(jnp.int32)
    idx_ref[0, 0, :] = idx

    @pl.when(i == 0)
    def _():
        cbn = cb - jnp.mean(cb, axis=1, keepdims=True)
        cbn_ref[...] = cbn / jnp.linalg.norm(cbn, axis=1, keepdims=True)
        hc_acc[0, 0] = hc_tile
        div_acc[...] = div_tile

    @pl.when(i > 0)
    def _():
        hc_acc[0, 0] += hc_tile
        div_acc[...] += div_tile

    @pl.when(i == nsteps - 1)
    def _():
        h_clust = -(hc_acc[0, 0] / n_rows)
        div = div_acc[...] / n_rows
        h_div = -jnp.sum(div * jnp.log2(div + 1e-8))
        loss_ref[0, 0] = h_clust - _GAMMA * h_div


def _make_sc_gather(n_rows):
    info = plsc.get_sparse_core_info()
    nw = info.num_cores * info.num_subcores
    b_per_w = n_rows // nw
    mesh = plsc.VectorSubcoreMesh(core_axis_name="c", subcore_axis_name="s")

    @functools.partial(
        pl.kernel, mesh=mesh,
        out_type=jax.ShapeDtypeStruct((n_rows, _D), jnp.float32),
        scratch_types=[
            pltpu.VMEM((b_per_w,), jnp.int32),
            pltpu.VMEM((b_per_w, _D), jnp.float32),
            pltpu.SemaphoreType.DMA,
        ],
    )
    def gather(table_hbm, idx_hbm, out_hbm, idx_v, rows_v, sem):
        wid = lax.axis_index("s") * info.num_cores + lax.axis_index("c")
        base = wid * b_per_w
        pltpu.sync_copy(idx_hbm.at[pl.ds(base, b_per_w)], idx_v)
        pltpu.async_copy(table_hbm.at[idx_v], rows_v, sem).wait()
        pltpu.sync_copy(rows_v, out_hbm.at[pl.ds(base, b_per_w)])

    return gather


def kernel(inputs, codebook):
    b, t, d = inputs.shape
    n = b * t
    nsteps = n // _ROW_TILE
    x = inputs.reshape(n, d)

    idx3, loss, cbn = pl.pallas_call(
        _vq_kernel,
        grid=(nsteps,),
        in_specs=[
            pl.BlockSpec((_ROW_TILE, _D), lambda i: (i, 0)),
            pl.BlockSpec((_K, _D), lambda i: (0, 0)),
        ],
        out_specs=[
            pl.BlockSpec((1, 1, _ROW_TILE), lambda i: (i, 0, 0)),
            pl.BlockSpec((1, 1), lambda i: (0, 0),
                         memory_space=pltpu.SMEM),
            pl.BlockSpec((_K, _D), lambda i: (0, 0)),
        ],
        out_shape=[
            jax.ShapeDtypeStruct((nsteps, 1, _ROW_TILE), jnp.int32),
            jax.ShapeDtypeStruct((1, 1), jnp.float32),
            jax.ShapeDtypeStruct((_K, _D), jnp.float32),
        ],
        scratch_shapes=[
            pltpu.SMEM((1, 1), jnp.float32),
            pltpu.VMEM((1, _K), jnp.float32),
        ],
        compiler_params=pltpu.CompilerParams(
            dimension_semantics=("arbitrary",)),
    )(x, codebook)

    idx_flat = idx3.reshape(n)
    quant = _make_sc_gather(n)(cbn, idx_flat)

    quantized = quant.reshape(1, b, t, d)
    nn_idx = idx3.reshape(b, t)
    qloss = jnp.full((b, t, _K), loss[0, 0], dtype=jnp.float32)
    return (quantized, qloss, nn_idx, codebook)
